# Initial kernel scaffold; baseline (speedup 1.0000x reference)
import jax
import jax.numpy as jnp
from jax.experimental import pallas as pl


def kernel(image, centroids, cluster_labels):
    raise NotImplementedError("write your pallas kernel here")


# speedup vs baseline: 126.3490x; 126.3490x over previous
import functools

import jax
import jax.numpy as jnp
from jax import lax
from jax.experimental import pallas as pl
from jax.experimental.pallas import tpu as pltpu
from jax.experimental.pallas import tpu_sc as plsc

BS = 32
P = 196
D = 32
K = 512
PS = 16
NROW = 14
IMG = 224
PPAD = 224
RPI = IMG * NROW
IDX_PAD = RPI + PS
NCHUNK = 28
CH = RPI // NCHUNK


def _assign_body(x_ref, c_ref, o_ref):
    c = c_ref[...]
    cn = jnp.sum(c * c, axis=0)
    x = x_ref[0]
    s = cn[None, :] - 2.0 * lax.dot_general(
        x, c, (((1,), (0,)), ((), ())),
        preferred_element_type=jnp.float32,
        precision=lax.Precision.HIGHEST)
    o_ref[0, 0, :] = jnp.argmax(s, axis=1).astype(jnp.int32)


def _transpose_body(l_ref, o_ref):
    o_ref[...] = l_ref[...].T


def _sc_gather_body(a_hbm, t_hbm, out_hbm, a_v, idx_v, rows_v, sem):
    wid = lax.axis_index("s") * 2 + lax.axis_index("c")
    pltpu.sync_copy(a_hbm.at[wid], a_v)
    for r in range(NROW):
        base = a_v[pl.ds(r * NROW, 16)] * 16
        for i in range(PS):
            off = (r * PS + i) * NROW
            idx_v[pl.ds(off, 16)] = base + i
    for j in range(NCHUNK):
        pltpu.async_copy(
            t_hbm.at[idx_v.at[pl.ds(j * CH, CH)]],
            rows_v.at[pl.ds(j * CH, CH)], sem).wait()
    pltpu.sync_copy(rows_v, out_hbm.at[wid])


@functools.cache
def _sc_gather():
    return pl.kernel(
        _sc_gather_body,
        out_type=jax.ShapeDtypeStruct((BS, RPI, PS), jnp.int32),
        mesh=plsc.VectorSubcoreMesh(core_axis_name="c", subcore_axis_name="s"),
        compiler_params=pltpu.CompilerParams(use_tc_tiling_on_sc=False),
        scratch_types=[
            pltpu.VMEM((PPAD,), jnp.int32),
            pltpu.VMEM((IDX_PAD,), jnp.int32),
            pltpu.VMEM((RPI, PS), jnp.int32),
            pltpu.SemaphoreType.DMA,
        ],
    )


def kernel(image, centroids, cluster_labels):
    img_p = jnp.pad(image, ((0, 0), (0, PPAD - P), (0, 0)))
    assign = pl.pallas_call(
        _assign_body,
        grid=(BS,),
        in_specs=[
            pl.BlockSpec((1, PPAD, D), lambda b: (b, 0, 0)),
            pl.BlockSpec((D, K), lambda b: (0, 0)),
        ],
        out_specs=pl.BlockSpec((1, 1, PPAD), lambda b: (b, 0, 0)),
        out_shape=jax.ShapeDtypeStruct((BS, 1, PPAD), jnp.int32),
    )(img_p, centroids)
    labels_t = pl.pallas_call(
        _transpose_body,
        out_shape=jax.ShapeDtypeStruct((K, PS * PS), jnp.int32),
    )(cluster_labels)
    out = _sc_gather()(assign.reshape(BS, PPAD),
                       labels_t.reshape(K * PS, PS))
    return out.reshape(BS, IMG, IMG)
